# baseline (speedup 1.0000x reference)
import jax
import jax.numpy as jnp
from jax.experimental import pallas as pl
from jax.experimental.pallas import tpu as pltpu

_STAGE_ROWS = 4096


def kernel(ref_tensor, table):
    batch, _ = ref_tensor.shape
    dim = table.shape[1]
    n_copies = batch // _STAGE_ROWS

    def body(table_ref, out_ref, stage, sem):
        stage[:, :] = jnp.broadcast_to(table_ref[:, :], stage.shape)
        copies = [
            pltpu.make_async_copy(
                stage, out_ref.at[pl.ds(i * _STAGE_ROWS, _STAGE_ROWS)], sem
            )
            for i in range(n_copies)
        ]
        for cp in copies:
            cp.start()
        for cp in copies:
            cp.wait()

    return pl.pallas_call(
        body,
        in_specs=[pl.BlockSpec(memory_space=pltpu.VMEM)],
        out_specs=pl.BlockSpec(memory_space=pltpu.MemorySpace.HBM),
        out_shape=jax.ShapeDtypeStruct((batch, dim), table.dtype),
        scratch_shapes=[
            pltpu.VMEM((_STAGE_ROWS, dim), jnp.float32),
            pltpu.SemaphoreType.DMA,
        ],
    )(table)


# speedup vs baseline: 1.0325x; 1.0325x over previous
import jax
import jax.numpy as jnp
from jax.experimental import pallas as pl
from jax.experimental.pallas import tpu as pltpu

_STAGE_ROWS = 256


def kernel(ref_tensor, table):
    batch, _ = ref_tensor.shape
    dim = table.shape[1]
    n_copies = batch // _STAGE_ROWS

    def body(table_ref, out_ref, stage, sem):
        stage[:, :] = jnp.broadcast_to(table_ref[:, :], stage.shape)
        copies = [
            pltpu.make_async_copy(
                stage, out_ref.at[pl.ds(i * _STAGE_ROWS, _STAGE_ROWS)], sem
            )
            for i in range(n_copies)
        ]
        for cp in copies:
            cp.start()
        for cp in copies:
            cp.wait()

    return pl.pallas_call(
        body,
        in_specs=[pl.BlockSpec(memory_space=pltpu.VMEM)],
        out_specs=pl.BlockSpec(memory_space=pltpu.MemorySpace.HBM),
        out_shape=jax.ShapeDtypeStruct((batch, dim), table.dtype),
        scratch_shapes=[
            pltpu.VMEM((_STAGE_ROWS, dim), jnp.float32),
            pltpu.SemaphoreType.DMA,
        ],
    )(table)
